# baseline (speedup 1.0000x reference)
import jax
import jax.numpy as jnp
from jax import lax
from jax.experimental import pallas as pl
from jax.experimental.pallas import tpu as pltpu
from jax.experimental.pallas import tpu_sc as plsc

SEP = 3
T = 8192
DIM = 1024
NSEG = 64
NC = 2
NS = 16
L = 16
NW = NC * NS
TPW = T // NW
NV = TPW // L
CHUNK = 8
NCHUNK = TPW // CHUNK
NVD = DIM // L
NCU = 16
NBUF = 4


def _bc(x):
    return lax.broadcast_in_dim(jnp.asarray(x, jnp.int32), (L,), ())


def _body(tok_tab, seg_tab, tokens, cu, out,
          tok_v, seg_tab_v, shared_seg, shared_tok, cnt_v, pref_v, seg_v,
          cu_v, excl_v, bufs, gsems, osems, sem_tab):
    wid = lax.axis_index("s") * NC + lax.axis_index("c")
    base = wid * TPW

    @pl.when(lax.axis_index("s") == 0)
    def _():
        pltpu.sync_copy(tokens, shared_tok)
        pltpu.sync_copy(seg_tab, shared_seg)

    plsc.subcore_barrier()
    pltpu.sync_copy(shared_tok, tok_v)
    tab_copy = pltpu.async_copy(shared_seg, seg_tab_v.at[pl.ds(0, NSEG)],
                                sem_tab)
    pltpu.sync_copy(cu, cu_v)
    ncu_real = cu.shape[0]

    def gather_desc(c, b):
        idx = tok_v.at[pl.ds(base + c * CHUNK, CHUNK)]
        return pltpu.make_async_copy(tok_tab.at[idx], bufs[b], gsems[b])

    def out_desc(c, b):
        return pltpu.make_async_copy(
            bufs[b], out.at[pl.ds(base + c * CHUNK, CHUNK)], osems[b])

    for c in range(NBUF - 1):
        gather_desc(c, c).start()

    lane = lax.iota(jnp.int32, L)
    m15 = lane == _bc(L - 1)
    sep_c = _bc(SEP)
    one_c = _bc(1)
    zero_c = _bc(0)

    def ph_a(bk, _):
        acc = zero_c
        for u in range(L):
            v = tok_v[pl.ds(bk * TPW + u * L, L)]
            cnt = plsc.all_reduce_population_count(v == sep_c)
            acc = jnp.where(lane == _bc(u), cnt, acc)
        cnt_v[pl.ds(bk * L, L)] = acc
        return 0

    lax.fori_loop(0, wid + 1, ph_a, 0)

    def ph_b(m, carry):
        cv = cnt_v[pl.ds(m * L, L)]
        inc = plsc.cumsum(cv) + _bc(carry)
        pref_v[pl.ds(m * L, L)] = inc - cv
        return jnp.max(inc)

    lax.fori_loop(0, wid + 1, ph_b, jnp.int32(0))

    last = _bc(ncu_real - 1)
    cuv = jnp.where(lane <= last,
                    plsc.load_gather(cu_v, [jnp.minimum(lane, last)]),
                    _bc(T))

    for k in range(NCU):
        s = cuv[k]
        sm1 = jnp.maximum(s - 1, 0)
        q = sm1 // L
        r = sm1 % L
        tv = plsc.load_gather(tok_v, [_bc(q * L) + lane])
        head = plsc.all_reduce_population_count(
            (tv == sep_c) & (lane <= _bc(r)))
        pref_q = plsc.load_gather(pref_v, [_bc(q)])
        excl_k = jnp.where(_bc(s) == zero_c, zero_c, pref_q + head)
        plsc.store_scatter(excl_v, [_bc(k)], excl_k, mask=m15)

    cu_s = [_bc(cuv[k]) for k in range(NCU)]

    def ph_c(i, _):
        j = wid * NV + i
        v = tok_v[pl.ds(j * L, L)]
        cs_g = (plsc.cumsum(jnp.where(v == sep_c, one_c, zero_c))
                + plsc.load_gather(pref_v, [_bc(j)]))
        t = lane + _bc(base + i * L)
        rid = zero_c
        for k in range(NCU):
            rid = rid + jnp.where(t >= cu_s[k], one_c, zero_c)
        rid = rid - one_c
        seg = cs_g - plsc.load_gather(excl_v, [rid])
        seg_v[pl.ds(i * L, L)] = jnp.minimum(seg, _bc(NSEG))
        return 0

    lax.fori_loop(0, NV, ph_c, 0)

    nan_c = lax.broadcast_in_dim(jnp.float32(jnp.nan), (L,), ())

    def ph_nan(v, _):
        seg_tab_v[NSEG, pl.ds(v * L, L)] = nan_c
        return 0

    lax.fori_loop(0, NVD, ph_nan, 0)
    tab_copy.wait()

    def group_body(g, _):
        rows_per_group = NBUF * CHUNK
        sgv = [seg_v[pl.ds(g * rows_per_group + h * L, L)]
               for h in range(rows_per_group // L)]
        sg_scalars = [sgv[rr // L][rr % L] for rr in range(rows_per_group)]
        for b in range(NBUF):
            c = g * NBUF + b
            gather_desc(c, b).wait()
            buf = bufs[b]

            for r in range(CHUNK):
                rr = b * CHUNK + r
                sg_r = sg_scalars[rr]

                @plsc.parallel_loop(0, DIM, step=L, unroll=8)
                def _(col, r=r, sg_r=sg_r):
                    plsc.addupdate(buf.at[r, pl.ds(col, L)],
                                   seg_tab_v[sg_r, pl.ds(col, L)])
            out_desc(c, b).start()

            @pl.when(c + NBUF - 1 < NCHUNK)
            def _(c=c, b=b):
                @pl.when(c >= 1)
                def _():
                    out_desc(c - 1, (b + NBUF - 1) % NBUF).wait()
                gather_desc(c + NBUF - 1, (b + NBUF - 1) % NBUF).start()
        return 0

    lax.fori_loop(0, NCHUNK // NBUF, group_body, 0)
    for c in range(NCHUNK - NBUF, NCHUNK):
        out_desc(c, c % NBUF).wait()


@jax.jit
def kernel(tokens, cu_seqlens, tok_table, seg_table):
    mesh = plsc.VectorSubcoreMesh(core_axis_name="c", subcore_axis_name="s",
                                  num_cores=NC, num_subcores=NS)
    run = pl.kernel(
        _body,
        out_type=jax.ShapeDtypeStruct((T, DIM), jnp.float32),
        mesh=mesh,
        compiler_params=pltpu.CompilerParams(needs_layout_passes=False),
        scratch_types=[
            pltpu.VMEM((T,), jnp.int32),
            pltpu.VMEM((NSEG + 1, DIM), jnp.float32),
            pltpu.VMEM_SHARED((NSEG, DIM), jnp.float32),
            pltpu.VMEM_SHARED((T,), jnp.int32),
            pltpu.VMEM((T // L,), jnp.int32),
            pltpu.VMEM((T // L,), jnp.int32),
            pltpu.VMEM((TPW,), jnp.int32),
            pltpu.VMEM((cu_seqlens.shape[0],), jnp.int32),
            pltpu.VMEM((NCU,), jnp.int32),
            [pltpu.VMEM((CHUNK, DIM), jnp.float32) for _ in range(NBUF)],
            [pltpu.SemaphoreType.DMA for _ in range(NBUF)],
            [pltpu.SemaphoreType.DMA for _ in range(NBUF)],
            pltpu.SemaphoreType.DMA,
        ],
    )
    return run(tok_table, seg_table, tokens, cu_seqlens)


# speedup vs baseline: 1.0003x; 1.0003x over previous
import jax
import jax.numpy as jnp
from jax import lax
from jax.experimental import pallas as pl
from jax.experimental.pallas import tpu as pltpu
from jax.experimental.pallas import tpu_sc as plsc

SEP = 3
T = 8192
DIM = 1024
NSEG = 64
NC = 2
NS = 16
L = 16
NW = NC * NS
TPW = T // NW
NV = TPW // L
CHUNK = 8
NCHUNK = TPW // CHUNK
NVD = DIM // L
NCU = 16
NBUF = 4


def _bc(x):
    return lax.broadcast_in_dim(jnp.asarray(x, jnp.int32), (L,), ())


def _body(tok_tab, seg_tab, tokens, cu, out,
          tok_v, seg_tab_v, shared_seg, cnt_v, pref_v, seg_v,
          cu_v, excl_v, bufs, gsems, osems, sem_tab):
    wid = lax.axis_index("s") * NC + lax.axis_index("c")
    base = wid * TPW

    @pl.when(lax.axis_index("s") == 0)
    def _():
        pltpu.sync_copy(seg_tab, shared_seg)

    plsc.subcore_barrier()
    tab_copy = pltpu.async_copy(shared_seg, seg_tab_v.at[pl.ds(0, NSEG)],
                                sem_tab)
    pltpu.sync_copy(tokens, tok_v)
    pltpu.sync_copy(cu, cu_v)
    ncu_real = cu.shape[0]

    def gather_desc(c, b):
        idx = tok_v.at[pl.ds(base + c * CHUNK, CHUNK)]
        return pltpu.make_async_copy(tok_tab.at[idx], bufs[b], gsems[b])

    def out_desc(c, b):
        return pltpu.make_async_copy(
            bufs[b], out.at[pl.ds(base + c * CHUNK, CHUNK)], osems[b])

    for c in range(NBUF - 1):
        gather_desc(c, c).start()

    lane = lax.iota(jnp.int32, L)
    m15 = lane == _bc(L - 1)
    sep_c = _bc(SEP)
    one_c = _bc(1)
    zero_c = _bc(0)

    def ph_a(bk, _):
        acc = zero_c
        for u in range(L):
            v = tok_v[pl.ds(bk * TPW + u * L, L)]
            cnt = plsc.all_reduce_population_count(v == sep_c)
            acc = jnp.where(lane == _bc(u), cnt, acc)
        cnt_v[pl.ds(bk * L, L)] = acc
        return 0

    lax.fori_loop(0, wid + 1, ph_a, 0)

    def ph_b(m, carry):
        cv = cnt_v[pl.ds(m * L, L)]
        inc = plsc.cumsum(cv) + _bc(carry)
        pref_v[pl.ds(m * L, L)] = inc - cv
        return jnp.max(inc)

    lax.fori_loop(0, wid + 1, ph_b, jnp.int32(0))

    last = _bc(ncu_real - 1)
    cuv = jnp.where(lane <= last,
                    plsc.load_gather(cu_v, [jnp.minimum(lane, last)]),
                    _bc(T))

    for k in range(NCU):
        s = cuv[k]
        sm1 = jnp.maximum(s - 1, 0)
        q = sm1 // L
        r = sm1 % L
        tv = plsc.load_gather(tok_v, [_bc(q * L) + lane])
        head = plsc.all_reduce_population_count(
            (tv == sep_c) & (lane <= _bc(r)))
        pref_q = plsc.load_gather(pref_v, [_bc(q)])
        excl_k = jnp.where(_bc(s) == zero_c, zero_c, pref_q + head)
        plsc.store_scatter(excl_v, [_bc(k)], excl_k, mask=m15)

    cu_s = [_bc(cuv[k]) for k in range(NCU)]

    def ph_c(i, _):
        j = wid * NV + i
        v = tok_v[pl.ds(j * L, L)]
        cs_g = (plsc.cumsum(jnp.where(v == sep_c, one_c, zero_c))
                + plsc.load_gather(pref_v, [_bc(j)]))
        t = lane + _bc(base + i * L)
        rid = zero_c
        for k in range(NCU):
            rid = rid + jnp.where(t >= cu_s[k], one_c, zero_c)
        rid = rid - one_c
        seg = cs_g - plsc.load_gather(excl_v, [rid])
        seg_v[pl.ds(i * L, L)] = jnp.minimum(seg, _bc(NSEG))
        return 0

    lax.fori_loop(0, NV, ph_c, 0)

    nan_c = lax.broadcast_in_dim(jnp.float32(jnp.nan), (L,), ())

    def ph_nan(v, _):
        seg_tab_v[NSEG, pl.ds(v * L, L)] = nan_c
        return 0

    lax.fori_loop(0, NVD, ph_nan, 0)
    tab_copy.wait()

    def group_body(g, _):
        rows_per_group = NBUF * CHUNK
        sgv = [seg_v[pl.ds(g * rows_per_group + h * L, L)]
               for h in range(rows_per_group // L)]
        sg_scalars = [sgv[rr // L][rr % L] for rr in range(rows_per_group)]
        for b in range(NBUF):
            c = g * NBUF + b
            gather_desc(c, b).wait()
            buf = bufs[b]

            for r in range(CHUNK):
                rr = b * CHUNK + r
                sg_r = sg_scalars[rr]

                @plsc.parallel_loop(0, DIM, step=L, unroll=8)
                def _(col, r=r, sg_r=sg_r):
                    plsc.addupdate(buf.at[r, pl.ds(col, L)],
                                   seg_tab_v[sg_r, pl.ds(col, L)])
            out_desc(c, b).start()

            @pl.when(c + NBUF - 1 < NCHUNK)
            def _(c=c, b=b):
                @pl.when(c >= 1)
                def _():
                    out_desc(c - 1, (b + NBUF - 1) % NBUF).wait()
                gather_desc(c + NBUF - 1, (b + NBUF - 1) % NBUF).start()
        return 0

    lax.fori_loop(0, NCHUNK // NBUF, group_body, 0)
    for c in range(NCHUNK - NBUF, NCHUNK):
        out_desc(c, c % NBUF).wait()


@jax.jit
def kernel(tokens, cu_seqlens, tok_table, seg_table):
    mesh = plsc.VectorSubcoreMesh(core_axis_name="c", subcore_axis_name="s",
                                  num_cores=NC, num_subcores=NS)
    run = pl.kernel(
        _body,
        out_type=jax.ShapeDtypeStruct((T, DIM), jnp.float32),
        mesh=mesh,
        compiler_params=pltpu.CompilerParams(needs_layout_passes=False),
        scratch_types=[
            pltpu.VMEM((T,), jnp.int32),
            pltpu.VMEM((NSEG + 1, DIM), jnp.float32),
            pltpu.VMEM_SHARED((NSEG, DIM), jnp.float32),
            pltpu.VMEM((T // L,), jnp.int32),
            pltpu.VMEM((T // L,), jnp.int32),
            pltpu.VMEM((TPW,), jnp.int32),
            pltpu.VMEM((cu_seqlens.shape[0],), jnp.int32),
            pltpu.VMEM((NCU,), jnp.int32),
            [pltpu.VMEM((CHUNK, DIM), jnp.float32) for _ in range(NBUF)],
            [pltpu.SemaphoreType.DMA for _ in range(NBUF)],
            [pltpu.SemaphoreType.DMA for _ in range(NBUF)],
            pltpu.SemaphoreType.DMA,
        ],
    )
    return run(tok_table, seg_table, tokens, cu_seqlens)
